# baseline (speedup 1.0000x reference)
import functools
import jax
import jax.numpy as jnp
from jax import lax
from jax.experimental import pallas as pl
from jax.experimental.pallas import tpu as pltpu, tpu_sc as plsc


def kernel(x, table):
    bsz, seq_len = x.shape
    ctx, dim = table.shape
    tbl = table[:seq_len]

    info = plsc.get_sparse_core_info()
    NC, NS = info.num_cores, info.num_subcores
    NW = NC * NS
    rows_per_w = seq_len // NW

    mesh = plsc.VectorSubcoreMesh(core_axis_name="c", subcore_axis_name="s")

    @functools.partial(
        pl.kernel,
        out_type=jax.ShapeDtypeStruct((bsz, seq_len, dim), table.dtype),
        mesh=mesh,
        scratch_types=[pltpu.SemaphoreType.DMA],
    )
    def k(tbl_hbm, out_hbm, sem):
        wid = lax.axis_index("s") * NC + lax.axis_index("c")
        base = wid * rows_per_w
        copies = []
        for b in range(bsz):
            copies.append(
                pltpu.make_async_copy(
                    tbl_hbm.at[pl.ds(base, rows_per_w)],
                    out_hbm.at[b, pl.ds(base, rows_per_w)],
                    sem,
                )
            )
        for c in copies:
            c.start()
        for c in copies:
            c.wait()

    return k(tbl)


# speedup vs baseline: 54.6300x; 54.6300x over previous
import functools
import jax
import jax.numpy as jnp
from jax import lax
from jax.experimental import pallas as pl
from jax.experimental.pallas import tpu as pltpu, tpu_sc as plsc


def kernel(x, table):
    bsz, seq_len = x.shape
    ctx, dim = table.shape
    tbl = table[:seq_len]

    info = plsc.get_sparse_core_info()
    NC, NS = info.num_cores, info.num_subcores
    NW = NC * NS
    rows_per_w = seq_len // NW
    CH = 32
    NCHUNK = rows_per_w // CH

    mesh = plsc.VectorSubcoreMesh(core_axis_name="c", subcore_axis_name="s")

    @functools.partial(
        pl.kernel,
        out_type=jax.ShapeDtypeStruct((bsz, seq_len, dim), table.dtype),
        mesh=mesh,
        scratch_types=[
            pltpu.VMEM((2, CH, dim), jnp.float32),
            pltpu.SemaphoreType.DMA,
            pltpu.SemaphoreType.DMA((2,)),
        ],
    )
    def k(tbl_hbm, out_hbm, buf, rsem, wsem):
        wid = lax.axis_index("s") * NC + lax.axis_index("c")
        base = wid * rows_per_w

        def read(i, slot):
            return pltpu.make_async_copy(
                tbl_hbm.at[pl.ds(base + i * CH, CH)], buf.at[slot], rsem
            )

        def writes(i, slot):
            return [
                pltpu.make_async_copy(
                    buf.at[slot],
                    out_hbm.at[b, pl.ds(base + i * CH, CH)],
                    wsem.at[slot],
                )
                for b in range(bsz)
            ]

        read(0, 0).start()
        for i in range(NCHUNK):
            cur = i % 2
            read(i, cur).wait()
            if i >= 1:
                for w in writes(i - 1, 1 - cur):
                    w.wait()
            if i + 1 < NCHUNK:
                read(i + 1, 1 - cur).start()
            for w in writes(i, cur):
                w.start()
        for w in writes(NCHUNK - 1, (NCHUNK - 1) % 2):
            w.wait()

    return k(tbl)


# speedup vs baseline: 55.3402x; 1.0130x over previous
import functools
import jax
import jax.numpy as jnp
from jax import lax
from jax.experimental import pallas as pl
from jax.experimental.pallas import tpu as pltpu, tpu_sc as plsc


def kernel(x, table):
    bsz, seq_len = x.shape
    ctx, dim = table.shape
    tbl = table[:seq_len]

    info = plsc.get_sparse_core_info()
    NC, NS = info.num_cores, info.num_subcores
    NW = NC * NS
    rows_per_w = seq_len // NW
    CH = 32
    NCHUNK = rows_per_w // CH
    NBUF = 3

    mesh = plsc.VectorSubcoreMesh(core_axis_name="c", subcore_axis_name="s")

    @functools.partial(
        pl.kernel,
        out_type=jax.ShapeDtypeStruct((bsz, seq_len, dim), table.dtype),
        mesh=mesh,
        scratch_types=[
            pltpu.VMEM((NBUF, CH, dim), jnp.float32),
            pltpu.SemaphoreType.DMA,
            pltpu.SemaphoreType.DMA((NBUF,)),
        ],
    )
    def k(tbl_hbm, out_hbm, buf, rsem, wsem):
        wid = lax.axis_index("s") * NC + lax.axis_index("c")
        base = wid * rows_per_w

        def read(i, slot):
            return pltpu.make_async_copy(
                tbl_hbm.at[pl.ds(base + i * CH, CH)], buf.at[slot], rsem
            )

        def writes(i, slot):
            return [
                pltpu.make_async_copy(
                    buf.at[slot],
                    out_hbm.at[b, pl.ds(base + i * CH, CH)],
                    wsem.at[slot],
                )
                for b in range(bsz)
            ]

        read(0, 0).start()
        if NCHUNK > 1:
            read(1, 1).start()
        for i in range(NCHUNK):
            cur = i % NBUF
            read(i, cur).wait()
            if i + 2 < NCHUNK:
                nxt = (i + 2) % NBUF
                if i >= 1:
                    for w in writes(i - 1, nxt):
                        w.wait()
                read(i + 2, nxt).start()
            for w in writes(i, cur):
                w.start()
        for i in (NCHUNK - 3, NCHUNK - 2, NCHUNK - 1):
            if i >= 0:
                for w in writes(i, i % NBUF):
                    w.wait()

    return k(tbl)
